# baseline (speedup 1.0000x reference)
import functools

import jax
import jax.numpy as jnp
from jax.experimental import pallas as pl
from jax.experimental.pallas import tpu as pltpu

_MAX_REL = 32
_TABLE = 2 * _MAX_REL + 1


def _rpe_kernel(x_ref, w_ref, o_ref, *, tile_r, batch, seq_len):
    r0 = pl.program_id(0) * tile_r
    r = r0 + jax.lax.broadcasted_iota(jnp.int32, (tile_r, _TABLE), 0)
    i = r // batch
    k = jax.lax.broadcasted_iota(jnp.int32, (tile_r, _TABLE), 1)
    pos = i + k - _MAX_REL
    interior = ((pos >= 0) & (pos < seq_len)).astype(jnp.float32)
    low = jnp.maximum(i - (_MAX_REL - 1), 0).astype(jnp.float32)
    high = jnp.maximum((seq_len - _MAX_REL) - i, 0).astype(jnp.float32)
    c = jnp.where(k == 0, low, jnp.where(k == _TABLE - 1, high, interior))
    bias = jnp.dot(c, w_ref[...], preferred_element_type=jnp.float32)
    o_ref[...] = x_ref[...] + bias * (1.0 / seq_len)


def kernel(x, W):
    seq_len, batch, d_model = x.shape
    rows = seq_len * batch
    tile_r = rows // 4
    x2 = x.reshape(rows, d_model)
    out = pl.pallas_call(
        functools.partial(
            _rpe_kernel, tile_r=tile_r, batch=batch, seq_len=seq_len
        ),
        grid=(rows // tile_r,),
        in_specs=[
            pl.BlockSpec((tile_r, d_model), lambda s: (s, 0)),
            pl.BlockSpec((_TABLE, d_model), lambda s: (0, 0)),
        ],
        out_specs=pl.BlockSpec((tile_r, d_model), lambda s: (s, 0)),
        out_shape=jax.ShapeDtypeStruct((rows, d_model), x.dtype),
        compiler_params=pltpu.CompilerParams(
            dimension_semantics=("parallel",),
        ),
    )(x2, W)
    return out.reshape(seq_len, batch, d_model)


# speedup vs baseline: 1.2098x; 1.2098x over previous
import functools

import jax
import jax.numpy as jnp
from jax.experimental import pallas as pl
from jax.experimental.pallas import tpu as pltpu

_MAX_REL = 32
_TABLE = 2 * _MAX_REL + 1


def _rpe_kernel(x_ref, w_ref, o_ref, *, tile_s, seq_len):
    s0 = pl.program_id(0) * tile_s
    i = s0 + jax.lax.broadcasted_iota(jnp.int32, (tile_s, _TABLE), 0)
    k = jax.lax.broadcasted_iota(jnp.int32, (tile_s, _TABLE), 1)
    pos = i + k - _MAX_REL
    interior = ((pos >= 0) & (pos < seq_len)).astype(jnp.float32)
    low = jnp.maximum(i - (_MAX_REL - 1), 0).astype(jnp.float32)
    high = jnp.maximum((seq_len - _MAX_REL) - i, 0).astype(jnp.float32)
    c = jnp.where(k == 0, low, jnp.where(k == _TABLE - 1, high, interior))
    c = c * (1.0 / seq_len)
    bias = jnp.dot(c, w_ref[...], preferred_element_type=jnp.float32)
    o_ref[...] = x_ref[...] + bias[:, None, :]


def kernel(x, W):
    seq_len, batch, d_model = x.shape
    tile_s = seq_len // 2
    out = pl.pallas_call(
        functools.partial(_rpe_kernel, tile_s=tile_s, seq_len=seq_len),
        grid=(seq_len // tile_s,),
        in_specs=[
            pl.BlockSpec((tile_s, batch, d_model), lambda s: (s, 0, 0)),
            pl.BlockSpec((_TABLE, d_model), lambda s: (0, 0)),
        ],
        out_specs=pl.BlockSpec((tile_s, batch, d_model), lambda s: (s, 0, 0)),
        out_shape=jax.ShapeDtypeStruct((seq_len, batch, d_model), x.dtype),
        compiler_params=pltpu.CompilerParams(
            dimension_semantics=("parallel",),
        ),
    )(x, W)
    return out
